# Initial kernel scaffold; baseline (speedup 1.0000x reference)
import jax
import jax.numpy as jnp
from jax.experimental import pallas as pl


def kernel(tile_values, value_table, pos_table):
    raise NotImplementedError("write your pallas kernel here")


# speedup vs baseline: 2.2638x; 2.2638x over previous
import functools

import jax
import jax.numpy as jnp
from jax import lax
from jax.experimental import pallas as pl
from jax.experimental.pallas import tpu as pltpu
from jax.experimental.pallas import tpu_sc as plsc

B = 1024
S = 1024
D = 64
NC, NS = 2, 16
NW = NC * NS
F = B * S
PER_W = F // NW
CHUNK = 128
NCHUNK = PER_W // CHUNK

_mesh = plsc.VectorSubcoreMesh(
    core_axis_name="c", subcore_axis_name="s", num_cores=NC, num_subcores=NS
)


@functools.partial(
    pl.kernel,
    out_type=jax.ShapeDtypeStruct((F, D), jnp.float32),
    mesh=_mesh,
    scratch_types=[
        pltpu.VMEM((S, D), jnp.float32),
        pltpu.VMEM((CHUNK,), jnp.int32),
        pltpu.VMEM((CHUNK, D), jnp.float32),
        pltpu.SemaphoreType.DMA,
    ],
    compiler_params=pltpu.CompilerParams(use_tc_tiling_on_sc=False),
)
def _embed(tv_hbm, table_hbm, pos_hbm, out_hbm, posbuf, idxbuf, rows, sem):
    wid = lax.axis_index("s") * NC + lax.axis_index("c")
    base = wid * PER_W
    pltpu.sync_copy(pos_hbm, posbuf)

    @pl.loop(0, NCHUNK)
    def _chunk(g):
        flat0 = base + g * CHUNK
        s0 = lax.rem(flat0, S)
        pltpu.sync_copy(tv_hbm.at[pl.ds(flat0, CHUNK)], idxbuf)
        pltpu.async_copy(table_hbm.at[idxbuf], rows, sem).wait()

        @pl.loop(0, CHUNK)
        def _row(i):
            for j in range(D // 16):
                plsc.addupdate(
                    rows.at[i, pl.ds(j * 16, 16)],
                    posbuf[s0 + i, pl.ds(j * 16, 16)],
                )

        pltpu.sync_copy(rows, out_hbm.at[pl.ds(flat0, CHUNK)])


def kernel(tile_values, value_table, pos_table):
    tv_flat = tile_values.reshape(F).astype(jnp.int32)
    out = _embed(tv_flat, value_table, pos_table)
    return out.reshape(B, S, D)


# speedup vs baseline: 2.6587x; 1.1744x over previous
import functools

import jax
import jax.numpy as jnp
from jax import lax
from jax.experimental import pallas as pl
from jax.experimental.pallas import tpu as pltpu
from jax.experimental.pallas import tpu_sc as plsc

B = 1024
S = 1024
D = 64
NC, NS = 2, 16
NW = NC * NS
F = B * S
PER_W = F // NW
CHUNK = 128
NCHUNK = PER_W // CHUNK

_mesh = plsc.VectorSubcoreMesh(
    core_axis_name="c", subcore_axis_name="s", num_cores=NC, num_subcores=NS
)


@functools.partial(
    pl.kernel,
    out_type=jax.ShapeDtypeStruct((F, D), jnp.float32),
    mesh=_mesh,
    scratch_types=[
        pltpu.VMEM((CHUNK,), jnp.int32),
        pltpu.VMEM((CHUNK, D), jnp.float32),
        pltpu.SemaphoreType.DMA,
    ],
    compiler_params=pltpu.CompilerParams(use_tc_tiling_on_sc=False),
)
def _embed(tv_hbm, table_hbm, pos_hbm, out_hbm, idxbuf, rows, sem):
    wid = lax.axis_index("s") * NC + lax.axis_index("c")
    base = wid * PER_W

    @pl.loop(0, NCHUNK)
    def _chunk(g):
        flat0 = base + g * CHUNK
        s0 = lax.rem(flat0, S)
        pltpu.sync_copy(tv_hbm.at[pl.ds(flat0, CHUNK)], idxbuf)
        pltpu.sync_copy(pos_hbm.at[pl.ds(s0, CHUNK)], rows)
        pltpu.async_copy(table_hbm.at[idxbuf], rows, sem, add=True).wait()
        pltpu.sync_copy(rows, out_hbm.at[pl.ds(flat0, CHUNK)])


def kernel(tile_values, value_table, pos_table):
    tv_flat = tile_values.reshape(F).astype(jnp.int32)
    out = _embed(tv_flat, value_table, pos_table)
    return out.reshape(B, S, D)


# speedup vs baseline: 3.6107x; 1.3581x over previous
import functools

import jax
import jax.numpy as jnp
from jax import lax
from jax.experimental import pallas as pl
from jax.experimental.pallas import tpu as pltpu
from jax.experimental.pallas import tpu_sc as plsc

B = 1024
S = 1024
D = 64
NC, NS = 2, 16
NW = NC * NS
F = B * S
PER_W = F // NW
CHUNK = 512
SUBG = CHUNK // 128
NCHUNK = PER_W // CHUNK

_mesh = plsc.VectorSubcoreMesh(
    core_axis_name="c", subcore_axis_name="s", num_cores=NC, num_subcores=NS
)


@functools.partial(
    pl.kernel,
    out_type=jax.ShapeDtypeStruct((F, D), jnp.float32),
    mesh=_mesh,
    scratch_types=[
        pltpu.VMEM((SUBG, 128), jnp.int32),
        pltpu.VMEM((SUBG, 128), jnp.int32),
        pltpu.VMEM((CHUNK, D), jnp.float32),
        pltpu.VMEM((CHUNK, D), jnp.float32),
        pltpu.SemaphoreType.DMA,
        pltpu.SemaphoreType.DMA,
        pltpu.SemaphoreType.DMA,
        pltpu.SemaphoreType.DMA,
        pltpu.SemaphoreType.DMA,
        pltpu.SemaphoreType.DMA,
    ],
    compiler_params=pltpu.CompilerParams(use_tc_tiling_on_sc=False),
)
def _embed(tv_hbm, table_hbm, pos_hbm, out_hbm,
           idx0, idx1, rows0, rows1, sip0, sip1, sg0, sg1, so0, so1):
    idx = (idx0, idx1)
    rows = (rows0, rows1)
    sip = (sip0, sip1)
    sg = (sg0, sg1)
    so = (so0, so1)
    wid = lax.axis_index("s") * NC + lax.axis_index("c")
    base = wid * PER_W

    def start(g, b):
        flat0 = base + g * CHUNK
        row0 = flat0 // 128
        s0 = lax.rem(flat0, S)
        pltpu.async_copy(tv_hbm.at[pl.ds(row0, SUBG), :], idx[b], sip[b])
        pltpu.async_copy(pos_hbm.at[pl.ds(s0, CHUNK)], rows[b], sip[b])

    def wait_ip(b):
        pltpu.make_async_copy(tv_hbm.at[pl.ds(0, SUBG), :], idx[b], sip[b]).wait()
        pltpu.make_async_copy(pos_hbm.at[pl.ds(0, CHUNK)], rows[b], sip[b]).wait()

    def fire_gathers(b):
        for j in range(SUBG):
            pltpu.async_copy(
                table_hbm.at[idx[b].at[j]],
                rows[b].at[pl.ds(j * 128, 128)],
                sg[b],
                add=True,
            )

    def wait_g(b):
        pltpu.make_async_copy(out_hbm.at[pl.ds(0, CHUNK)], rows[b], sg[b]).wait()

    def fire_out(g, b):
        flat0 = base + g * CHUNK
        pltpu.async_copy(rows[b], out_hbm.at[pl.ds(flat0, CHUNK)], so[b])

    def wait_out(b):
        pltpu.make_async_copy(rows[b], out_hbm.at[pl.ds(0, CHUNK)], so[b]).wait()

    start(0, 0)
    start(1, 1)
    wait_ip(0)
    fire_gathers(0)

    @pl.loop(0, NCHUNK, step=2)
    def _go(go):
        wait_ip(1)
        fire_gathers(1)
        wait_g(0)
        fire_out(go, 0)
        wait_out(0)

        @pl.when(go + 2 < NCHUNK)
        def _():
            start(go + 2, 0)

        wait_g(1)
        fire_out(go + 1, 1)
        wait_out(1)

        @pl.when(go + 3 < NCHUNK)
        def _():
            start(go + 3, 1)

        @pl.when(go + 2 < NCHUNK)
        def _():
            wait_ip(0)
            fire_gathers(0)


def kernel(tile_values, value_table, pos_table):
    tv2 = tile_values.reshape(F // 128, 128).astype(jnp.int32)
    out = _embed(tv2, value_table, pos_table)
    return out.reshape(B, S, D)


# speedup vs baseline: 3.6184x; 1.0021x over previous
import functools

import jax
import jax.numpy as jnp
from jax import lax
from jax.experimental import pallas as pl
from jax.experimental.pallas import tpu as pltpu
from jax.experimental.pallas import tpu_sc as plsc

B = 1024
S = 1024
D = 64
NC, NS = 2, 16
NW = NC * NS
F = B * S
PER_W = F // NW
CHUNK = 256
SUBG = CHUNK // 128
NCHUNK = PER_W // CHUNK
NBUF = 4

_mesh = plsc.VectorSubcoreMesh(
    core_axis_name="c", subcore_axis_name="s", num_cores=NC, num_subcores=NS
)


@functools.partial(
    pl.kernel,
    out_type=jax.ShapeDtypeStruct((F, D), jnp.float32),
    mesh=_mesh,
    scratch_types=(
        [pltpu.VMEM((SUBG, 128), jnp.int32) for _ in range(NBUF)]
        + [pltpu.VMEM((CHUNK, D), jnp.float32) for _ in range(NBUF)]
        + [pltpu.SemaphoreType.DMA for _ in range(3 * NBUF)]
    ),
    compiler_params=pltpu.CompilerParams(use_tc_tiling_on_sc=False),
)
def _embed(tv_hbm, table_hbm, pos_hbm, out_hbm, *scratch):
    idx = scratch[:NBUF]
    rows = scratch[NBUF:2 * NBUF]
    sip = scratch[2 * NBUF:3 * NBUF]
    sg = scratch[3 * NBUF:4 * NBUF]
    so = scratch[4 * NBUF:5 * NBUF]
    wid = lax.axis_index("s") * NC + lax.axis_index("c")
    base = wid * PER_W

    def start(g, b):
        flat0 = base + g * CHUNK
        row0 = flat0 // 128
        s0 = lax.rem(flat0, S)
        pltpu.async_copy(tv_hbm.at[pl.ds(row0, SUBG), :], idx[b], sip[b])
        pltpu.async_copy(pos_hbm.at[pl.ds(s0, CHUNK)], rows[b], sip[b])

    def wait_ip(b):
        pltpu.make_async_copy(tv_hbm.at[pl.ds(0, SUBG), :], idx[b], sip[b]).wait()
        pltpu.make_async_copy(pos_hbm.at[pl.ds(0, CHUNK)], rows[b], sip[b]).wait()

    def fire_gathers(b):
        for j in range(SUBG):
            pltpu.async_copy(
                table_hbm.at[idx[b].at[j]],
                rows[b].at[pl.ds(j * 128, 128)],
                sg[b],
                add=True,
            )

    def wait_g(b):
        pltpu.make_async_copy(out_hbm.at[pl.ds(0, CHUNK)], rows[b], sg[b]).wait()

    def fire_out(g, b):
        flat0 = base + g * CHUNK
        pltpu.async_copy(rows[b], out_hbm.at[pl.ds(flat0, CHUNK)], so[b])

    def wait_out(b):
        pltpu.make_async_copy(rows[b], out_hbm.at[pl.ds(0, CHUNK)], so[b]).wait()

    for b in range(NBUF):
        start(b, b)
    for b in range(NBUF):
        wait_ip(b)
        fire_gathers(b)

    @pl.loop(0, NCHUNK, step=NBUF)
    def _go(go):
        for b in range(NBUF):
            wait_g(b)
            fire_out(go + b, b)
        for b in range(NBUF):
            gnxt = go + NBUF + b

            @pl.when(gnxt < NCHUNK)
            def _(b=b, gnxt=gnxt):
                wait_out(b)
                start(gnxt, b)

        for b in range(NBUF):

            @pl.when(go + NBUF + b < NCHUNK)
            def _(b=b):
                wait_ip(b)
                fire_gathers(b)

    for b in range(NBUF):
        wait_out(b)


def kernel(tile_values, value_table, pos_table):
    tv2 = tile_values.reshape(F // 128, 128).astype(jnp.int32)
    out = _embed(tv2, value_table, pos_table)
    return out.reshape(B, S, D)
